# Initial kernel scaffold; baseline (speedup 1.0000x reference)
import jax
import jax.numpy as jnp
from jax.experimental import pallas as pl


def kernel(x_node_features, edge_index, edge_weight, root_indices_in_batch, batch_vector, W1, b1, W2, b2, Wl, bl):
    raise NotImplementedError("write your pallas kernel here")


# speedup vs baseline: 2.9245x; 2.9245x over previous
import functools

import jax
import jax.numpy as jnp
from jax import lax
from jax.experimental import pallas as pl
from jax.experimental.pallas import tpu as pltpu
from jax.experimental.pallas import tpu_sc as plsc

N, E, D, G = 10000, 320000, 128, 64
NC, NS, L = 2, 16, 16
NW = NC * NS
CHUNK = 128
GROUP = 8
EPW = 10240
EPAD = NW * EPW
NPAD = 10240
RPT = NPAD // NS
NGROUPS = EPW // (CHUNK * GROUP)


def _sc_scatter_body(xw_hbm, srcr_hbm, dstr_hbm, wr_hbm, out_hbm,
                     src_v, dst_v, w_v, rows_v, accum, sem):
    c = lax.axis_index("c")
    s = lax.axis_index("s")
    wid = s * NC + c

    zero = jnp.zeros((L,), jnp.float32)

    def _zrow(r, carry):
        for k in range(D // L):
            rows_v[r, pl.ds(k * L, L)] = zero
        return carry

    lax.fori_loop(0, CHUNK, _zrow, 0)
    base_row = s * RPT
    for t in range(RPT // CHUNK):
        pltpu.sync_copy(rows_v, accum.at[pl.ds(base_row + t * CHUNK, CHUNK), :])
    plsc.subcore_barrier()

    row0 = wid * (EPW // CHUNK)

    def _group(g, carry):
        pltpu.sync_copy(srcr_hbm.at[pl.ds(row0 + g * GROUP, GROUP), :], src_v)
        pltpu.sync_copy(dstr_hbm.at[pl.ds(row0 + g * GROUP, GROUP), :], dst_v)
        pltpu.sync_copy(wr_hbm.at[pl.ds(row0 + g * GROUP, GROUP), :], w_v)
        for j in range(GROUP):
            pltpu.sync_copy(xw_hbm.at[src_v.at[j]], rows_v)

            def _mulblk(rb, cc):
                base = rb * L
                wvec = w_v[j, pl.ds(base, L)]
                for l in range(L):
                    ws = wvec[l]
                    for k in range(D // L):
                        sl = pl.ds(k * L, L)
                        rows_v[base + l, sl] = rows_v[base + l, sl] * ws
                return cc

            lax.fori_loop(0, CHUNK // L, _mulblk, 0)
            pltpu.sync_copy(rows_v, accum.at[dst_v.at[j]], add=True)
        return carry

    lax.fori_loop(0, NGROUPS, _group, 0)
    plsc.subcore_barrier()

    for t in range(RPT // CHUNK):
        sl = pl.ds(base_row + t * CHUNK, CHUNK)
        pltpu.sync_copy(accum.at[sl, :], rows_v)
        pltpu.sync_copy(rows_v, out_hbm.at[c, sl, :])


_sc_scatter = pl.kernel(
    _sc_scatter_body,
    out_type=jax.ShapeDtypeStruct((NC, NPAD, D), jnp.float32),
    mesh=plsc.VectorSubcoreMesh(core_axis_name="c", subcore_axis_name="s",
                                num_cores=NC, num_subcores=NS),
    scratch_types=[
        pltpu.VMEM((GROUP, CHUNK), jnp.int32),
        pltpu.VMEM((GROUP, CHUNK), jnp.int32),
        pltpu.VMEM((GROUP, CHUNK), jnp.float32),
        pltpu.VMEM((CHUNK, D), jnp.float32),
        pltpu.VMEM_SHARED((NPAD, D), jnp.float32),
        pltpu.SemaphoreType.DMA,
    ],
)


BR_A = 2000
BR_B = 1024
BR_C = 2000


def _leaky(x):
    return jnp.where(x > 0, x, jnp.float32(0.01) * x)


def _tca_body(x_ref, w1_ref, w2b_ref, roots_ref, xw1_ref, rx_ref, acc_ref):
    i = pl.program_id(0)
    xb = x_ref[...]
    xw1_ref[...] = jnp.dot(xb, w1_ref[...], preferred_element_type=jnp.float32)
    rows = lax.broadcasted_iota(jnp.int32, (G, BR_A), 1) + i * BR_A
    oh = (roots_ref[...] == rows).astype(jnp.float32)
    contrib = jnp.dot(oh, xb, preferred_element_type=jnp.float32)
    acc_ref[...] = jnp.where(i == 0, contrib, acc_ref[...] + contrib)

    @pl.when(i == pl.num_programs(0) - 1)
    def _():
        rx_ref[...] = jnp.dot(acc_ref[...], w2b_ref[...],
                              preferred_element_type=jnp.float32)


def _tcb_body(p0_ref, p1_ref, b1_ref, w2a_ref, wlb_ref, rx_ref, batch_ref,
              roots_ref, xw2_ref, rh_ref, acc_ref):
    i = pl.program_id(0)
    h1 = p0_ref[...] + p1_ref[...] + b1_ref[...]
    rows = lax.broadcasted_iota(jnp.int32, (G, BR_B), 1) + i * BR_B
    oh_r = (roots_ref[...] == rows).astype(jnp.float32)
    contrib = jnp.dot(oh_r, h1, preferred_element_type=jnp.float32)
    acc_ref[...] = jnp.where(i == 0, contrib, acc_ref[...] + contrib)
    cols = lax.broadcasted_iota(jnp.int32, (BR_B, G), 1)
    oh_b = (batch_ref[...] == cols).astype(jnp.float32)
    xw2_ref[...] = (
        jnp.dot(_leaky(h1), w2a_ref[...], preferred_element_type=jnp.float32)
        + jnp.dot(oh_b, rx_ref[...], preferred_element_type=jnp.float32))

    @pl.when(i == pl.num_programs(0) - 1)
    def _():
        rh_ref[...] = jnp.dot(acc_ref[...], wlb_ref[...],
                              preferred_element_type=jnp.float32)


def _tcc_body(p0_ref, p1_ref, b2_ref, wla_ref, rh_ref, bl_ref, batch_ref,
              out_ref):
    h2a = _leaky(p0_ref[...] + p1_ref[...] + b2_ref[...])
    cols = lax.broadcasted_iota(jnp.int32, (BR_C, G), 1)
    oh_b = (batch_ref[...] == cols).astype(jnp.float32)
    out_ref[...] = _leaky(
        jnp.dot(h2a, wla_ref[...], preferred_element_type=jnp.float32)
        + jnp.dot(oh_b, rh_ref[...], preferred_element_type=jnp.float32)
        + bl_ref[...])


def _row_spec(br, d):
    return pl.BlockSpec((br, d), lambda i: (i, 0))


def _full_spec(shape):
    return pl.BlockSpec(shape, lambda i: tuple(0 for _ in shape))


_tca = pl.pallas_call(
    _tca_body,
    grid=(N // BR_A,),
    in_specs=[_row_spec(BR_A, D), _full_spec((D, D)), _full_spec((D, D)),
              _full_spec((G, 1))],
    out_specs=[_row_spec(BR_A, D), _full_spec((G, D))],
    out_shape=[jax.ShapeDtypeStruct((N, D), jnp.float32),
               jax.ShapeDtypeStruct((G, D), jnp.float32)],
    scratch_shapes=[pltpu.VMEM((G, D), jnp.float32)],
)

_tcb = pl.pallas_call(
    _tcb_body,
    grid=(NPAD // BR_B,),
    in_specs=[_row_spec(BR_B, D), _row_spec(BR_B, D), _full_spec((1, D)),
              _full_spec((D, D)), _full_spec((D, D)), _full_spec((G, D)),
              _row_spec(BR_B, 1), _full_spec((G, 1))],
    out_specs=[_row_spec(BR_B, D), _full_spec((G, D))],
    out_shape=[jax.ShapeDtypeStruct((NPAD, D), jnp.float32),
               jax.ShapeDtypeStruct((G, D), jnp.float32)],
    scratch_shapes=[pltpu.VMEM((G, D), jnp.float32)],
)

_tcc = pl.pallas_call(
    _tcc_body,
    grid=(N // BR_C,),
    in_specs=[_row_spec(BR_C, D), _row_spec(BR_C, D), _full_spec((1, D)),
              _full_spec((D, D)), _full_spec((G, D)), _full_spec((1, D)),
              _row_spec(BR_C, 1)],
    out_specs=_row_spec(BR_C, D),
    out_shape=jax.ShapeDtypeStruct((N, D), jnp.float32),
)


def kernel(x_node_features, edge_index, edge_weight, root_indices_in_batch,
           batch_vector, W1, b1, W2, b2, Wl, bl):
    src = edge_index[0].astype(jnp.int32)
    dst = edge_index[1].astype(jnp.int32)
    pad = EPAD - E
    srcr = jnp.concatenate([src, jnp.zeros((pad,), jnp.int32)]).reshape(
        EPAD // CHUNK, CHUNK)
    dstr = jnp.concatenate([dst, jnp.zeros((pad,), jnp.int32)]).reshape(
        EPAD // CHUNK, CHUNK)
    wr = jnp.concatenate(
        [edge_weight, jnp.zeros((pad,), jnp.float32)]).reshape(
        EPAD // CHUNK, CHUNK)

    roots = root_indices_in_batch.astype(jnp.int32).reshape(G, 1)
    batch_p = jnp.concatenate(
        [batch_vector.astype(jnp.int32),
         jnp.zeros((NPAD - N,), jnp.int32)]).reshape(NPAD, 1)

    W2a, W2b = W2[:D], W2[D:]
    Wla, Wlb = Wl[:D], Wl[D:]
    b1r = b1.reshape(1, D)
    b2r = b2.reshape(1, D)
    blr = bl.reshape(1, D)

    xw1, rx = _tca(x_node_features, W1, W2b, roots)
    part1 = _sc_scatter(xw1, srcr, dstr, wr)
    xw2, rh = _tcb(part1[0], part1[1], b1r, W2a, Wlb, rx, batch_p, roots)
    part2 = _sc_scatter(xw2, srcr, dstr, wr)
    out = _tcc(part2[0][:N, :], part2[1][:N, :], b2r, Wla, rh,
               blr, batch_p[:N])
    return out
